# baseline (speedup 1.0000x reference)
import functools

import jax
import jax.numpy as jnp
from jax import lax
from jax.experimental import pallas as pl
from jax.experimental.pallas import tpu as pltpu
from jax.experimental.pallas import tpu_sc as plsc

_B = 16384
_NUM_TYPES = 64
_HALF = 64
_D = 128

_NC = 2
_NS = 16
_NW = _NC * _NS
_BPW = _B // _NW
_CHUNK = 128
_NCHUNK = _BPW // _CHUNK

@functools.cache
def _make_rel_gather():
    mesh = plsc.VectorSubcoreMesh(
        core_axis_name="c", subcore_axis_name="s", num_cores=_NC, num_subcores=_NS
    )

    @functools.partial(
        pl.kernel,
        out_type=jax.ShapeDtypeStruct((_NW, _BPW, _HALF), jnp.float32),
        mesh=mesh,
        scratch_types=[
            pltpu.VMEM((_NCHUNK, _CHUNK), jnp.int32),
            pltpu.VMEM((_BPW, _HALF), jnp.float32),
            pltpu.SemaphoreType.DMA,
        ],
        compiler_params=pltpu.CompilerParams(use_tc_tiling_on_sc=False),
    )
    def _rel_gather(rel_ids_hbm, rel_table_hbm, out_hbm, idx_v, rows_v, sem):
        wid = lax.axis_index("s") * _NC + lax.axis_index("c")
        pltpu.sync_copy(rel_ids_hbm.at[wid], idx_v)
        copies = []
        for j in range(_NCHUNK):
            copies.append(
                pltpu.async_copy(
                    rel_table_hbm.at[idx_v.at[j]],
                    rows_v.at[pl.ds(j * _CHUNK, _CHUNK)],
                    sem,
                )
            )
        for c in copies:
            c.wait()
        pltpu.sync_copy(rows_v, out_hbm.at[wid])

    return _rel_gather


def _encode_block(tid_ref, rel_ref, tt_ref, wt_ref, b_ref, g_ref, be_ref, o_ref):
    ids = tid_ref[0, 0, :]
    bm = ids.shape[0]
    onehot = (ids[:, None] == lax.broadcasted_iota(jnp.int32, (bm, _NUM_TYPES), 1)
              ).astype(jnp.float32)
    t_emb = jnp.dot(onehot, tt_ref[...], preferred_element_type=jnp.float32)
    wt = wt_ref[...]
    proj = (
        jnp.dot(t_emb, wt[:_HALF, :], preferred_element_type=jnp.float32)
        + jnp.dot(rel_ref[...], wt[_HALF:, :], preferred_element_type=jnp.float32)
        + b_ref[...]
    )
    mean = jnp.mean(proj, axis=-1, keepdims=True)
    cent = proj - mean
    var = jnp.mean(jnp.square(cent), axis=-1, keepdims=True)
    o_ref[...] = cent * lax.rsqrt(var + 1e-5) * g_ref[...] + be_ref[...]


_BM = 1024
_GRID = _B // _BM


def kernel(type_ids, rel_ids, type_table, rel_table, W, b, gamma, beta):
    rel_ids_r = rel_ids.reshape(_NW, _NCHUNK, _CHUNK)
    rel_emb = _make_rel_gather()(rel_ids_r, rel_table).reshape(_B, _HALF)

    tid_r = type_ids.reshape(_GRID, 1, _BM)
    wt = W.T
    out = pl.pallas_call(
        _encode_block,
        grid=(_GRID,),
        in_specs=[
            pl.BlockSpec((1, 1, _BM), lambda i: (i, 0, 0)),
            pl.BlockSpec((_BM, _HALF), lambda i: (i, 0)),
            pl.BlockSpec((_NUM_TYPES, _HALF), lambda i: (0, 0)),
            pl.BlockSpec((_D, _D), lambda i: (0, 0)),
            pl.BlockSpec((1, _D), lambda i: (0, 0)),
            pl.BlockSpec((1, _D), lambda i: (0, 0)),
            pl.BlockSpec((1, _D), lambda i: (0, 0)),
        ],
        out_specs=pl.BlockSpec((_BM, _D), lambda i: (i, 0)),
        out_shape=jax.ShapeDtypeStruct((_B, _D), jnp.float32),
        compiler_params=pltpu.CompilerParams(
            dimension_semantics=("arbitrary",),
        ),
    )(tid_r, rel_emb, type_table, wt, b.reshape(1, _D),
      gamma.reshape(1, _D), beta.reshape(1, _D))
    return out


# speedup vs baseline: 1.6124x; 1.6124x over previous
import functools

import jax
import jax.numpy as jnp
from jax import lax
from jax.experimental import pallas as pl
from jax.experimental.pallas import tpu as pltpu
from jax.experimental.pallas import tpu_sc as plsc

_B = 16384
_NUM_TYPES = 64
_HALF = 64
_D = 128

_NC = 2
_NS = 16
_NW = _NC * _NS
_BPW = _B // _NW
_Q = 16
_NG = _BPW // _Q


@functools.cache
def _make_rel_gather():
    mesh = plsc.VectorSubcoreMesh(
        core_axis_name="c", subcore_axis_name="s", num_cores=_NC, num_subcores=_NS
    )

    @functools.partial(
        pl.kernel,
        out_type=jax.ShapeDtypeStruct((_NW, _BPW, _HALF), jnp.float32),
        mesh=mesh,
        scratch_types=[
            pltpu.VMEM((_BPW,), jnp.int32),
            pltpu.VMEM((_BPW, _HALF), jnp.float32),
            pltpu.SemaphoreType.DMA,
        ],
    )
    def _rel_gather(rel_ids_hbm, table_hbm, out_hbm, idx_v, rows_v, sem):
        wid = lax.axis_index("s") * _NC + lax.axis_index("c")
        pltpu.sync_copy(rel_ids_hbm.at[wid], idx_v)

        def group(g, _):
            base = g * _Q
            vec = idx_v[pl.ds(base, _Q)]
            copies = []
            for k in range(_Q):
                copies.append(
                    pltpu.async_copy(
                        table_hbm.at[vec[k]], rows_v.at[base + k], sem
                    )
                )
            for c in copies:
                c.wait()
            return _

        lax.fori_loop(0, _NG, group, 0)
        pltpu.sync_copy(rows_v, out_hbm.at[wid])

    return _rel_gather


def _encode_block(tid_ref, rel_ref, tt_ref, wt_ref, b_ref, g_ref, be_ref, o_ref):
    ids = tid_ref[0, 0, :]
    bm = ids.shape[0]
    onehot = (ids[:, None] == lax.broadcasted_iota(jnp.int32, (bm, _NUM_TYPES), 1)
              ).astype(jnp.float32)
    t_emb = jnp.dot(onehot, tt_ref[...], preferred_element_type=jnp.float32)
    wt = wt_ref[...]
    proj = (
        jnp.dot(t_emb, wt[:_HALF, :], preferred_element_type=jnp.float32)
        + jnp.dot(rel_ref[...], wt[_HALF:, :], preferred_element_type=jnp.float32)
        + b_ref[...]
    )
    mean = jnp.mean(proj, axis=-1, keepdims=True)
    cent = proj - mean
    var = jnp.mean(jnp.square(cent), axis=-1, keepdims=True)
    o_ref[...] = cent * lax.rsqrt(var + 1e-5) * g_ref[...] + be_ref[...]


_BM = 1024
_GRID = _B // _BM


def kernel(type_ids, rel_ids, type_table, rel_table, W, b, gamma, beta):
    rel_ids_r = rel_ids.reshape(_NW, _BPW)
    rel_emb = _make_rel_gather()(rel_ids_r, rel_table).reshape(_B, _HALF)

    tid_r = type_ids.reshape(_GRID, 1, _BM)
    wt = W.T
    out = pl.pallas_call(
        _encode_block,
        grid=(_GRID,),
        in_specs=[
            pl.BlockSpec((1, 1, _BM), lambda i: (i, 0, 0)),
            pl.BlockSpec((_BM, _HALF), lambda i: (i, 0)),
            pl.BlockSpec((_NUM_TYPES, _HALF), lambda i: (0, 0)),
            pl.BlockSpec((_D, _D), lambda i: (0, 0)),
            pl.BlockSpec((1, _D), lambda i: (0, 0)),
            pl.BlockSpec((1, _D), lambda i: (0, 0)),
            pl.BlockSpec((1, _D), lambda i: (0, 0)),
        ],
        out_specs=pl.BlockSpec((_BM, _D), lambda i: (i, 0)),
        out_shape=jax.ShapeDtypeStruct((_B, _D), jnp.float32),
        compiler_params=pltpu.CompilerParams(
            dimension_semantics=("arbitrary",),
        ),
    )(tid_r, rel_emb, type_table, wt, b.reshape(1, _D),
      gamma.reshape(1, _D), beta.reshape(1, _D))
    return out
